# Initial kernel scaffold; baseline (speedup 1.0000x reference)
import functools

import jax
import jax.numpy as jnp
from jax import lax
from jax.experimental import pallas as pl
from jax.experimental.pallas import tpu as pltpu
from jax.experimental.pallas import tpu_sc as plsc

_N = 10000
_E = 320000
_D_IN = 128
_D_HID = 32
_D_OUT = 16

_NC = 2
_NS = 16
_NW = _NC * _NS
_EPW = _E // _NW
_C = 125
_NCHUNK = _EPW // _C
_NBUF = 4
_N_PAD = 10240
_RPS = _N_PAD // _NS

_mesh = plsc.VectorSubcoreMesh(
    core_axis_name="c", subcore_axis_name="s", num_cores=_NC, num_subcores=_NS
)


@jax.jit
def _deg_sc(dst_w):
    @functools.partial(
        pl.kernel,
        out_type=jax.ShapeDtypeStruct((_NW, _N_PAD), jnp.float32),
        mesh=_mesh,
        scratch_types=[
            pltpu.VMEM((_EPW,), jnp.int32),
            pltpu.VMEM((_N_PAD,), jnp.float32),
        ],
    )
    def deg_kernel(dst_hbm, out_hbm, dst_v, deg_v):
        cid = lax.axis_index("c")
        sid = lax.axis_index("s")
        wid = sid * _NC + cid

        zeros16 = jnp.zeros((16,), jnp.float32)

        @pl.loop(0, _N_PAD, step=16)
        def _(i):
            deg_v[pl.ds(i, 16)] = zeros16

        pltpu.sync_copy(dst_hbm.at[wid], dst_v)
        ones16 = jnp.ones((16,), jnp.float32)

        @pl.loop(0, _EPW, step=16)
        def _(k):
            idx = dst_v[pl.ds(k, 16)]
            plsc.addupdate_scatter(deg_v, [idx], ones16)

        pltpu.sync_copy(deg_v, out_hbm.at[wid])

    return deg_kernel(dst_w)


@jax.jit
def _seg_sc(table, src_c, dst_c, zrows):
    @functools.partial(
        pl.kernel,
        out_type=jax.ShapeDtypeStruct((_NC, _N_PAD, _D_HID), jnp.float32),
        mesh=_mesh,
        scratch_types=[
            pltpu.VMEM((_NCHUNK, _C), jnp.int32),
            pltpu.VMEM((_NCHUNK, _C), jnp.int32),
            [pltpu.VMEM((_C, _D_HID), jnp.float32) for _ in range(_NBUF)],
            pltpu.VMEM_SHARED((_N_PAD, _D_HID), jnp.float32),
            [pltpu.SemaphoreType.DMA for _ in range(2 * _NBUF)],
        ],
    )
    def seg_kernel(table_hbm, src_hbm, dst_hbm, z_hbm, out_hbm,
                   src_v, dst_v, bufs, acc, sems):
        cid = lax.axis_index("c")
        sid = lax.axis_index("s")
        wid = sid * _NC + cid
        gsem, ssem = sems[:_NBUF], sems[_NBUF:]

        pltpu.sync_copy(src_hbm.at[wid], src_v)
        pltpu.sync_copy(dst_hbm.at[wid], dst_v)
        stripe = pl.ds(sid * _RPS, _RPS)
        pltpu.sync_copy(z_hbm.at[stripe], acc.at[stripe])
        plsc.subcore_barrier()

        for b in range(_NBUF):
            pltpu.async_copy(table_hbm.at[src_v.at[b]], bufs[b], gsem[b])

        @pl.loop(0, _NCHUNK, step=_NBUF)
        def _(j0):
            for b in range(_NBUF):
                j = j0 + b
                pltpu.make_async_copy(
                    table_hbm.at[src_v.at[j]], bufs[b], gsem[b]).wait()
                pltpu.async_copy(
                    bufs[b], acc.at[dst_v.at[j]], ssem[b], add=True)
                pltpu.make_async_copy(
                    bufs[b], acc.at[dst_v.at[j]], ssem[b]).wait()

                @pl.when(j + _NBUF < _NCHUNK)
                def _():
                    pltpu.async_copy(
                        table_hbm.at[src_v.at[j + _NBUF]], bufs[b], gsem[b])

        plsc.subcore_barrier()
        pltpu.sync_copy(acc.at[stripe], out_hbm.at[cid, stripe])

    return seg_kernel(table, src_c, dst_c, zrows)


@jax.jit
def _mm1_tc(x, w1):
    def body(x_ref, w_ref, o_ref):
        o_ref[...] = jnp.dot(x_ref[...], w_ref[...],
                             preferred_element_type=jnp.float32)

    return pl.pallas_call(
        body, out_shape=jax.ShapeDtypeStruct((_N, _D_HID), jnp.float32)
    )(x, w1)


@jax.jit
def _scale_tc(deg_parts, xw):
    def body(p_ref, xw_ref, dinv_ref, y1_ref):
        deg = jnp.sum(p_ref[...], axis=0) + 1.0
        dinv = lax.rsqrt(deg)
        dinv_ref[...] = dinv[:, None]
        y1_ref[...] = xw_ref[...] * dinv[:_N, None]

    return pl.pallas_call(
        body,
        out_shape=(
            jax.ShapeDtypeStruct((_N_PAD, 1), jnp.float32),
            jax.ShapeDtypeStruct((_N, _D_HID), jnp.float32),
        ),
    )(deg_parts, xw)


@jax.jit
def _hidden_tc(parts, y1, dinv, b1):
    def body(p_ref, y1_ref, dinv_ref, b1_ref, h_ref, y2_ref):
        agg = p_ref[0, :_N, :] + p_ref[1, :_N, :] + y1_ref[...]
        dv = dinv_ref[:_N, :]
        h = jnp.maximum(agg * dv + b1_ref[...], 0.0)
        h_ref[...] = h
        y2_ref[...] = h * dv

    return pl.pallas_call(
        body,
        out_shape=(
            jax.ShapeDtypeStruct((_N, _D_HID), jnp.float32),
            jax.ShapeDtypeStruct((_N, _D_HID), jnp.float32),
        ),
    )(parts, y1, dinv, b1.reshape(1, _D_HID))


@jax.jit
def _out_tc(parts, y2, dinv, wmu, bmu, wlv, blv):
    def body(p_ref, y2_ref, dinv_ref, wmu_ref, bmu_ref, wlv_ref, blv_ref,
             mu_ref, lv_ref):
        z = (p_ref[0, :_N, :] + p_ref[1, :_N, :] + y2_ref[...]) * dinv_ref[:_N, :]
        mu_ref[...] = jnp.dot(z, wmu_ref[...],
                              preferred_element_type=jnp.float32) + bmu_ref[...]
        lv_ref[...] = jnp.dot(z, wlv_ref[...],
                              preferred_element_type=jnp.float32) + blv_ref[...]

    return pl.pallas_call(
        body,
        out_shape=(
            jax.ShapeDtypeStruct((_N, _D_OUT), jnp.float32),
            jax.ShapeDtypeStruct((_N, _D_OUT), jnp.float32),
        ),
    )(parts, y2, dinv, wmu, bmu.reshape(1, _D_OUT), wlv, blv.reshape(1, _D_OUT))


def kernel(x, edge_index, W1, b1, Wmu, bmu, Wlv, blv):
    src = edge_index[0]
    dst = edge_index[1]
    src_c = src.reshape(_NW, _NCHUNK, _C)
    dst_c = dst.reshape(_NW, _NCHUNK, _C)
    dst_w = dst.reshape(_NW, _EPW)
    zrows = jnp.zeros((_N_PAD, _D_HID), jnp.float32)

    deg_parts = _deg_sc(dst_w)
    xw = _mm1_tc(x, W1)
    dinv, y1 = _scale_tc(deg_parts, xw)
    agg1 = _seg_sc(y1, src_c, dst_c, zrows)
    h, y2 = _hidden_tc(agg1, y1, dinv, b1)
    agg2 = _seg_sc(y2, src_c, dst_c, zrows)
    return _out_tc(agg2, y2, dinv, Wmu, bmu, Wlv, blv)


# speedup vs baseline: 65.3044x; 65.3044x over previous
import dataclasses
import functools

import jax
import jax.numpy as jnp
from jax import lax
from jax.experimental import pallas as pl
from jax.experimental.pallas import tpu as pltpu
from jax.experimental.pallas import tpu_sc as plsc

_N = 10000
_E = 320000
_D_IN = 128
_D_HID = 32
_D_OUT = 16

_NC = 2
_NS = 16
_NW = _NC * _NS
_EPW = _E // _NW
_C = 125
_NCHUNK = _EPW // _C
_NBUF = 4
_N_PAD = 10240
_RPS = _N_PAD // _NS

_mesh = plsc.VectorSubcoreMesh(
    core_axis_name="c", subcore_axis_name="s", num_cores=_NC, num_subcores=_NS
)

_cp = pltpu.CompilerParams()
if "needs_layout_passes" in pltpu.CompilerParams.__dataclass_fields__:
    _cp = dataclasses.replace(_cp, needs_layout_passes=False)
_cp_untiled = dataclasses.replace(_cp, use_tc_tiling_on_sc=False)


@jax.jit
def _deg_sc(dst_w):
    @functools.partial(
        pl.kernel,
        out_type=jax.ShapeDtypeStruct((_NW, _N_PAD), jnp.float32),
        mesh=_mesh,
        scratch_types=[
            pltpu.VMEM((_EPW,), jnp.int32),
            pltpu.VMEM((_N_PAD,), jnp.float32),
        ],
        compiler_params=_cp,
    )
    def deg_kernel(dst_hbm, out_hbm, dst_v, deg_v):
        cid = lax.axis_index("c")
        sid = lax.axis_index("s")
        wid = sid * _NC + cid

        zeros16 = jnp.zeros((16,), jnp.float32)

        @pl.loop(0, _N_PAD, step=16)
        def _(i):
            deg_v[pl.ds(i, 16)] = zeros16

        pltpu.sync_copy(dst_hbm.at[wid], dst_v)
        ones16 = jnp.ones((16,), jnp.float32)

        @pl.loop(0, _EPW, step=16)
        def _(k):
            idx = dst_v[pl.ds(k, 16)]
            plsc.addupdate_scatter(deg_v, [idx], ones16)

        pltpu.sync_copy(deg_v, out_hbm.at[wid])

    return deg_kernel(dst_w)


@jax.jit
def _seg_sc(table, src_c, dst_c, zrows):
    @functools.partial(
        pl.kernel,
        out_type=jax.ShapeDtypeStruct((_NC, _N_PAD, _D_HID), jnp.float32),
        mesh=_mesh,
        scratch_types=[
            pltpu.VMEM((_NCHUNK, _C), jnp.int32),
            pltpu.VMEM((_NCHUNK, _C), jnp.int32),
            [pltpu.VMEM((_C, _D_HID), jnp.float32) for _ in range(_NBUF)],
            pltpu.VMEM_SHARED((_N_PAD, _D_HID), jnp.float32),
            [pltpu.SemaphoreType.DMA for _ in range(2 * _NBUF)],
        ],
        compiler_params=_cp_untiled,
    )
    def seg_kernel(table_hbm, src_hbm, dst_hbm, z_hbm, out_hbm,
                   src_v, dst_v, bufs, acc, sems):
        cid = lax.axis_index("c")
        sid = lax.axis_index("s")
        wid = sid * _NC + cid
        gsem, ssem = sems[:_NBUF], sems[_NBUF:]

        pltpu.sync_copy(src_hbm.at[wid], src_v)
        pltpu.sync_copy(dst_hbm.at[wid], dst_v)
        stripe = pl.ds(sid * _RPS, _RPS)
        pltpu.sync_copy(z_hbm.at[stripe], acc.at[stripe])
        plsc.subcore_barrier()

        for b in range(_NBUF):
            pltpu.async_copy(table_hbm.at[src_v.at[b]], bufs[b], gsem[b])

        @pl.loop(0, _NCHUNK, step=_NBUF)
        def _(j0):
            for b in range(_NBUF):
                j = j0 + b
                pltpu.make_async_copy(
                    table_hbm.at[src_v.at[j]], bufs[b], gsem[b]).wait()
                pltpu.async_copy(
                    bufs[b], acc.at[dst_v.at[j]], ssem[b], add=True)
                pltpu.make_async_copy(
                    bufs[b], acc.at[dst_v.at[j]], ssem[b]).wait()

                @pl.when(j + _NBUF < _NCHUNK)
                def _():
                    pltpu.async_copy(
                        table_hbm.at[src_v.at[j + _NBUF]], bufs[b], gsem[b])

        plsc.subcore_barrier()
        pltpu.sync_copy(acc.at[stripe], out_hbm.at[cid, stripe])

    return seg_kernel(table, src_c, dst_c, zrows)


@jax.jit
def _mm1_tc(x, w1):
    def body(x_ref, w_ref, o_ref):
        o_ref[...] = jnp.dot(x_ref[...], w_ref[...],
                             preferred_element_type=jnp.float32)

    return pl.pallas_call(
        body, out_shape=jax.ShapeDtypeStruct((_N, _D_HID), jnp.float32)
    )(x, w1)


@jax.jit
def _scale_tc(deg_parts, xw):
    def body(p_ref, xw_ref, dinv_ref, y1_ref):
        deg = jnp.sum(p_ref[...], axis=0) + 1.0
        dinv = lax.rsqrt(deg)
        dinv_ref[...] = dinv[:, None]
        y1_ref[...] = xw_ref[...] * dinv[:_N, None]

    return pl.pallas_call(
        body,
        out_shape=(
            jax.ShapeDtypeStruct((_N_PAD, 1), jnp.float32),
            jax.ShapeDtypeStruct((_N, _D_HID), jnp.float32),
        ),
    )(deg_parts, xw)


@jax.jit
def _hidden_tc(parts, y1, dinv, b1):
    def body(p_ref, y1_ref, dinv_ref, b1_ref, h_ref, y2_ref):
        agg = p_ref[0, :_N, :] + p_ref[1, :_N, :] + y1_ref[...]
        dv = dinv_ref[:_N, :]
        h = jnp.maximum(agg * dv + b1_ref[...], 0.0)
        h_ref[...] = h
        y2_ref[...] = h * dv

    return pl.pallas_call(
        body,
        out_shape=(
            jax.ShapeDtypeStruct((_N, _D_HID), jnp.float32),
            jax.ShapeDtypeStruct((_N, _D_HID), jnp.float32),
        ),
    )(parts, y1, dinv, b1.reshape(1, _D_HID))


@jax.jit
def _out_tc(parts, y2, dinv, wmu, bmu, wlv, blv):
    def body(p_ref, y2_ref, dinv_ref, wmu_ref, bmu_ref, wlv_ref, blv_ref,
             mu_ref, lv_ref):
        z = (p_ref[0, :_N, :] + p_ref[1, :_N, :] + y2_ref[...]) * dinv_ref[:_N, :]
        mu_ref[...] = jnp.dot(z, wmu_ref[...],
                              preferred_element_type=jnp.float32) + bmu_ref[...]
        lv_ref[...] = jnp.dot(z, wlv_ref[...],
                              preferred_element_type=jnp.float32) + blv_ref[...]

    return pl.pallas_call(
        body,
        out_shape=(
            jax.ShapeDtypeStruct((_N, _D_OUT), jnp.float32),
            jax.ShapeDtypeStruct((_N, _D_OUT), jnp.float32),
        ),
    )(parts, y2, dinv, wmu, bmu.reshape(1, _D_OUT), wlv, blv.reshape(1, _D_OUT))


def kernel(x, edge_index, W1, b1, Wmu, bmu, Wlv, blv):
    src = edge_index[0]
    dst = edge_index[1]
    src_c = src.reshape(_NW, _NCHUNK, _C)
    dst_c = dst.reshape(_NW, _NCHUNK, _C)
    dst_w = dst.reshape(_NW, _EPW)
    zrows = jnp.zeros((_N_PAD, _D_HID), jnp.float32)

    deg_parts = _deg_sc(dst_w)
    xw = _mm1_tc(x, W1)
    dinv, y1 = _scale_tc(deg_parts, xw)
    agg1 = _seg_sc(y1, src_c, dst_c, zrows)
    h, y2 = _hidden_tc(agg1, y1, dinv, b1)
    agg2 = _seg_sc(y2, src_c, dst_c, zrows)
    return _out_tc(agg2, y2, dinv, Wmu, bmu, Wlv, blv)
